# Initial kernel scaffold; baseline (speedup 1.0000x reference)
import jax
import jax.numpy as jnp
from jax.experimental import pallas as pl


def kernel(x, batch):
    raise NotImplementedError("write your pallas kernel here")


# speedup vs baseline: 5.6081x; 5.6081x over previous
import functools

import jax
import jax.numpy as jnp
from jax import lax
from jax.experimental import pallas as pl
from jax.experimental.pallas import tpu as pltpu
from jax.experimental.pallas import tpu_sc as plsc

N = 100000
D = 128
S = 512
NC = 2
NS = 16
NW = NC * NS
BLK = 128
NB = N // BLK
TAIL = N - NB * BLK
SEG_PER_TILE = S // NS
BASE_BLOCKS = NB // NW
EXTRA_WORKERS = NB - BASE_BLOCKS * NW


def _sc_partials(x, batch):
    mesh = plsc.VectorSubcoreMesh(core_axis_name="c", subcore_axis_name="s")

    @functools.partial(
        pl.kernel,
        out_type=[
            jax.ShapeDtypeStruct((NC, S, D), jnp.float32),
            jax.ShapeDtypeStruct((NW, S), jnp.float32),
        ],
        mesh=mesh,
        compiler_params=pltpu.CompilerParams(needs_layout_passes=False),
        scratch_types=[
            pltpu.VMEM((BLK, D), jnp.float32),
            pltpu.VMEM((BLK,), jnp.int32),
            pltpu.VMEM((TAIL,), jnp.int32),
            pltpu.VMEM((S,), jnp.float32),
            pltpu.VMEM((SEG_PER_TILE, D), jnp.float32),
            pltpu.VMEM_SHARED((S, D), jnp.float32),
        ],
    )
    def sc_kernel(x_hbm, b_hbm, sum_out, cnt_out, xb, ids, ids_t, hist, zbuf, acc):
        c = lax.axis_index("c")
        s = lax.axis_index("s")
        wid = c * NS + s

        zeros16 = jnp.zeros((16,), jnp.float32)

        def zrow(i, carry):
            def zcol(j, carry2):
                zbuf[i, pl.ds(j * 16, 16)] = zeros16
                return carry2
            return lax.fori_loop(0, D // 16, zcol, carry)
        lax.fori_loop(0, SEG_PER_TILE, zrow, 0)

        def zh(i, carry):
            hist[pl.ds(i * 16, 16)] = zeros16
            return carry
        lax.fori_loop(0, S // 16, zh, 0)

        pltpu.sync_copy(zbuf, acc.at[pl.ds(s * SEG_PER_TILE, SEG_PER_TILE)])
        plsc.subcore_barrier()

        lane = lax.iota(jnp.int32, 16)
        ones = jnp.full((16,), 1.0, jnp.float32)

        def count_block(id_ref, nvec):
            def grp(g, carry):
                idv = id_ref[pl.ds(g * 16, 16)]
                for m in range(16):
                    plsc.addupdate_scatter(hist, [idv], ones, mask=lane == m)
                return carry
            lax.fori_loop(0, nvec, grp, 0)

        nblk = BASE_BLOCKS + jnp.where(wid < EXTRA_WORKERS, 1, 0)

        def body(k, carry):
            base = (wid + k * NW) * BLK
            pltpu.sync_copy(b_hbm.at[pl.ds(base, BLK)], ids)
            pltpu.sync_copy(x_hbm.at[pl.ds(base, BLK)], xb)
            pltpu.sync_copy(xb, acc.at[ids], add=True)
            count_block(ids, BLK // 16)
            return carry
        lax.fori_loop(0, nblk, body, 0)

        @pl.when(wid == NW - 1)
        def _tail():
            base = NB * BLK
            pltpu.sync_copy(b_hbm.at[pl.ds(base, TAIL)], ids_t)
            pltpu.sync_copy(x_hbm.at[pl.ds(base, TAIL)], xb.at[pl.ds(0, TAIL)])
            pltpu.sync_copy(xb.at[pl.ds(0, TAIL)], acc.at[ids_t], add=True)
            count_block(ids_t, TAIL // 16)

        pltpu.sync_copy(hist, cnt_out.at[wid])
        plsc.subcore_barrier()
        pltpu.sync_copy(acc.at[pl.ds(s * SEG_PER_TILE, SEG_PER_TILE)],
                        sum_out.at[c, pl.ds(s * SEG_PER_TILE, SEG_PER_TILE)])

    return sc_kernel(x, batch)


def _combine(partial_sums, partial_counts):
    def body(sp_ref, cn_ref, o_ref):
        total = sp_ref[0] + sp_ref[1]
        cnt = jnp.maximum(jnp.sum(cn_ref[...], axis=0), 1.0)
        o_ref[...] = total / cnt[:, None]

    return pl.pallas_call(
        body,
        out_shape=jax.ShapeDtypeStruct((S, D), jnp.float32),
    )(partial_sums, partial_counts)


def kernel(x, batch):
    batch = batch.astype(jnp.int32)
    partial_sums, partial_counts = _sc_partials(x, batch)
    return _combine(partial_sums, partial_counts)


# speedup vs baseline: 8.2333x; 1.4681x over previous
import functools

import jax
import jax.numpy as jnp
from jax import lax
from jax.experimental import pallas as pl
from jax.experimental.pallas import tpu as pltpu
from jax.experimental.pallas import tpu_sc as plsc

N = 100000
D = 128
S = 512
NC = 2
NS = 16
NW = NC * NS
BLK = 128
NB = N // BLK
TAIL = N - NB * BLK
SEG_PER_TILE = S // NS
BASE_BLOCKS = NB // NW
EXTRA_WORKERS = NB - BASE_BLOCKS * NW
MAXB = BASE_BLOCKS + 1


def _sc_partials(x, batch2d, batch_tail):
    mesh = plsc.VectorSubcoreMesh(core_axis_name="c", subcore_axis_name="s")

    @functools.partial(
        pl.kernel,
        out_type=[
            jax.ShapeDtypeStruct((NC, S, D), jnp.float32),
            jax.ShapeDtypeStruct((NW, S), jnp.float32),
        ],
        mesh=mesh,
        compiler_params=pltpu.CompilerParams(needs_layout_passes=False,
                                             use_tc_tiling_on_sc=False),
        scratch_types=[
            pltpu.VMEM((2, BLK, D), jnp.float32),
            pltpu.VMEM((MAXB, BLK), jnp.int32),
            pltpu.VMEM((TAIL, D), jnp.float32),
            pltpu.VMEM((TAIL,), jnp.int32),
            pltpu.VMEM((S,), jnp.float32),
            pltpu.VMEM((SEG_PER_TILE, D), jnp.float32),
            pltpu.VMEM_SHARED((S, D), jnp.float32),
            pltpu.SemaphoreType.DMA((2,)),
            pltpu.SemaphoreType.DMA,
        ],
    )
    def sc_kernel(x_hbm, b2d_hbm, btail_hbm, sum_out, cnt_out,
                  xbufs, ids_all, xt, ids_t, hist, zbuf, acc, ld_sems, sc_sem):
        c = lax.axis_index("c")
        s = lax.axis_index("s")
        wid = c * NS + s

        zeros16 = jnp.zeros((16,), jnp.float32)

        def zrow(i, carry):
            def zcol(j, carry2):
                zbuf[i, pl.ds(j * 16, 16)] = zeros16
                return carry2
            return lax.fori_loop(0, D // 16, zcol, carry)
        lax.fori_loop(0, SEG_PER_TILE, zrow, 0)

        def zh(i, carry):
            hist[pl.ds(i * 16, 16)] = zeros16
            return carry
        lax.fori_loop(0, S // 16, zh, 0)

        pltpu.sync_copy(zbuf, acc.at[pl.ds(s * SEG_PER_TILE, SEG_PER_TILE)])
        plsc.subcore_barrier()

        lane = lax.iota(jnp.int32, 16)
        ones = jnp.full((16,), 1.0, jnp.float32)

        sb = BASE_BLOCKS * wid + jnp.minimum(wid, EXTRA_WORKERS)
        nblk = BASE_BLOCKS + jnp.where(wid < EXTRA_WORKERS, 1, 0)

        pltpu.sync_copy(b2d_hbm.at[pl.ds(sb, MAXB)], ids_all)

        for p in range(2):
            pltpu.async_copy(x_hbm.at[pl.ds((sb + p) * BLK, BLK)],
                             xbufs.at[p], ld_sems.at[p])

        def pair(i, carry):
            for p in range(2):
                k = 2 * i + p

                @pl.when(k < nblk)
                def _block():
                    pltpu.make_async_copy(
                        x_hbm.at[pl.ds((sb + k) * BLK, BLK)],
                        xbufs.at[p], ld_sems.at[p]).wait()
                    h = pltpu.async_copy(xbufs.at[p], acc.at[ids_all.at[k]],
                                         sc_sem, add=True)

                    def grp(g, carry2):
                        idv = ids_all[k, pl.ds(g * 16, 16)]
                        for m in range(16):
                            plsc.addupdate_scatter(hist, [idv], ones,
                                                   mask=lane == m)
                        return carry2
                    lax.fori_loop(0, BLK // 16, grp, 0)
                    h.wait()

                    @pl.when(k + 2 < nblk)
                    def _next_load():
                        pltpu.async_copy(
                            x_hbm.at[pl.ds((sb + k + 2) * BLK, BLK)],
                            xbufs.at[p], ld_sems.at[p])
            return carry
        lax.fori_loop(0, (MAXB + 1) // 2, pair, 0)

        @pl.when(wid == NW - 1)
        def _tail():
            base = NB * BLK
            pltpu.sync_copy(btail_hbm, ids_t)
            pltpu.sync_copy(x_hbm.at[pl.ds(base, TAIL)], xt)
            pltpu.sync_copy(xt, acc.at[ids_t], add=True)

            def grp(g, carry):
                idv = ids_t[pl.ds(g * 16, 16)]
                for m in range(16):
                    plsc.addupdate_scatter(hist, [idv], ones, mask=lane == m)
                return carry
            lax.fori_loop(0, TAIL // 16, grp, 0)

        pltpu.sync_copy(hist, cnt_out.at[wid])
        plsc.subcore_barrier()
        pltpu.sync_copy(acc.at[pl.ds(s * SEG_PER_TILE, SEG_PER_TILE)],
                        sum_out.at[c, pl.ds(s * SEG_PER_TILE, SEG_PER_TILE)])

    return sc_kernel(x, batch2d, batch_tail)


def _combine(partial_sums, partial_counts):
    def body(sp_ref, cn_ref, o_ref):
        total = sp_ref[0] + sp_ref[1]
        cnt = jnp.maximum(jnp.sum(cn_ref[...], axis=0), 1.0)
        o_ref[...] = total / cnt[:, None]

    return pl.pallas_call(
        body,
        out_shape=jax.ShapeDtypeStruct((S, D), jnp.float32),
    )(partial_sums, partial_counts)


def kernel(x, batch):
    batch = batch.astype(jnp.int32)
    batch2d = jnp.concatenate(
        [batch[:NB * BLK], jnp.zeros((BLK,), jnp.int32)]).reshape(NB + 1, BLK)
    batch_tail = batch[NB * BLK:]
    partial_sums, partial_counts = _sc_partials(x, batch2d, batch_tail)
    return _combine(partial_sums, partial_counts)


# speedup vs baseline: 8.6234x; 1.0474x over previous
import functools

import jax
import jax.numpy as jnp
from jax import lax
from jax.experimental import pallas as pl
from jax.experimental.pallas import tpu as pltpu
from jax.experimental.pallas import tpu_sc as plsc

N = 100000
D = 128
S = 512
NC = 2
NS = 16
NW = NC * NS
BLK = 128
NB = N // BLK
TAIL = N - NB * BLK
SEG_PER_TILE = S // NS
BASE_BLOCKS = NB // NW
EXTRA_WORKERS = NB - BASE_BLOCKS * NW
MAXB = BASE_BLOCKS + 1


def _sc_partials(x, batch):
    mesh = plsc.VectorSubcoreMesh(core_axis_name="c", subcore_axis_name="s")

    @functools.partial(
        pl.kernel,
        out_type=[
            jax.ShapeDtypeStruct((NC, S, D), jnp.float32),
            jax.ShapeDtypeStruct((NW, S), jnp.float32),
        ],
        mesh=mesh,
        compiler_params=pltpu.CompilerParams(needs_layout_passes=False,
                                             use_tc_tiling_on_sc=False),
        scratch_types=[
            pltpu.VMEM((2, BLK, D), jnp.float32),
            pltpu.VMEM((MAXB, BLK), jnp.int32),
            pltpu.VMEM((TAIL, D), jnp.float32),
            pltpu.VMEM((TAIL,), jnp.int32),
            pltpu.VMEM((S,), jnp.float32),
            pltpu.VMEM((SEG_PER_TILE, D), jnp.float32),
            pltpu.VMEM_SHARED((S, D), jnp.float32),
            pltpu.SemaphoreType.DMA((2,)),
            pltpu.SemaphoreType.DMA,
            pltpu.SemaphoreType.DMA,
        ],
    )
    def sc_kernel(x_hbm, b_hbm, sum_out, cnt_out,
                  xbufs, ids_all, xt, ids_t, hist, zbuf, acc,
                  ld_sems, sc_sem, id_sem):
        c = lax.axis_index("c")
        s = lax.axis_index("s")
        wid = s * NC + c

        sb = BASE_BLOCKS * wid + jnp.minimum(wid, EXTRA_WORKERS)
        nblk = BASE_BLOCKS + jnp.where(wid < EXTRA_WORKERS, 1, 0)

        for k in range(MAXB):
            @pl.when(k < nblk)
            def _stage_ids():
                pltpu.async_copy(b_hbm.at[pl.ds((sb + k) * BLK, BLK)],
                                 ids_all.at[k], id_sem)

        for p in range(2):
            pltpu.async_copy(x_hbm.at[pl.ds((sb + p) * BLK, BLK)],
                             xbufs.at[p], ld_sems.at[p])

        zeros16 = jnp.zeros((16,), jnp.float32)

        def zrow(i, carry):
            def zcol(j, carry2):
                zbuf[i, pl.ds(j * 16, 16)] = zeros16
                return carry2
            return lax.fori_loop(0, D // 16, zcol, carry)
        lax.fori_loop(0, SEG_PER_TILE, zrow, 0)

        def zh(i, carry):
            hist[pl.ds(i * 16, 16)] = zeros16
            return carry
        lax.fori_loop(0, S // 16, zh, 0)

        pltpu.sync_copy(zbuf, acc.at[pl.ds(s * SEG_PER_TILE, SEG_PER_TILE)])
        plsc.subcore_barrier()

        for k in range(MAXB):
            @pl.when(k < nblk)
            def _drain_ids():
                pltpu.make_async_copy(b_hbm.at[pl.ds((sb + k) * BLK, BLK)],
                                      ids_all.at[k], id_sem).wait()

        lane = lax.iota(jnp.int32, 16)
        ones = jnp.full((16,), 1.0, jnp.float32)

        def pair(i, carry):
            for p in range(2):
                k = 2 * i + p

                @pl.when(k < nblk)
                def _block():
                    pltpu.make_async_copy(
                        x_hbm.at[pl.ds((sb + k) * BLK, BLK)],
                        xbufs.at[p], ld_sems.at[p]).wait()
                    h = pltpu.async_copy(xbufs.at[p], acc.at[ids_all.at[k]],
                                         sc_sem, add=True)

                    def grp(g, carry2):
                        idv = ids_all[k, pl.ds(g * 16, 16)]
                        for m in range(16):
                            plsc.addupdate_scatter(hist, [idv], ones,
                                                   mask=lane == m)
                        return carry2
                    lax.fori_loop(0, BLK // 16, grp, 0)
                    h.wait()

                    @pl.when(k + 2 < nblk)
                    def _next_load():
                        pltpu.async_copy(
                            x_hbm.at[pl.ds((sb + k + 2) * BLK, BLK)],
                            xbufs.at[p], ld_sems.at[p])
            return carry
        lax.fori_loop(0, (MAXB + 1) // 2, pair, 0)

        @pl.when(wid == NW - 1)
        def _tail():
            base = NB * BLK
            pltpu.sync_copy(b_hbm.at[pl.ds(base, TAIL)], ids_t)
            pltpu.sync_copy(x_hbm.at[pl.ds(base, TAIL)], xt)
            pltpu.sync_copy(xt, acc.at[ids_t], add=True)

            def grp(g, carry):
                idv = ids_t[pl.ds(g * 16, 16)]
                for m in range(16):
                    plsc.addupdate_scatter(hist, [idv], ones, mask=lane == m)
                return carry
            lax.fori_loop(0, TAIL // 16, grp, 0)

        pltpu.sync_copy(hist, cnt_out.at[wid])
        plsc.subcore_barrier()
        pltpu.sync_copy(acc.at[pl.ds(s * SEG_PER_TILE, SEG_PER_TILE)],
                        sum_out.at[c, pl.ds(s * SEG_PER_TILE, SEG_PER_TILE)])

    return sc_kernel(x, batch)


def _combine(partial_sums, partial_counts):
    def body(sp_ref, cn_ref, o_ref):
        total = sp_ref[0] + sp_ref[1]
        cnt = jnp.maximum(jnp.sum(cn_ref[...], axis=0), 1.0)
        o_ref[...] = total / cnt[:, None]

    return pl.pallas_call(
        body,
        out_shape=jax.ShapeDtypeStruct((S, D), jnp.float32),
    )(partial_sums, partial_counts)


def kernel(x, batch):
    batch = batch.astype(jnp.int32)
    partial_sums, partial_counts = _sc_partials(x, batch)
    return _combine(partial_sums, partial_counts)


# speedup vs baseline: 8.7050x; 1.0095x over previous
import functools

import jax
import jax.numpy as jnp
from jax import lax
from jax.experimental import pallas as pl
from jax.experimental.pallas import tpu as pltpu
from jax.experimental.pallas import tpu_sc as plsc

N = 100000
D = 128
S = 512
NC = 2
NS = 16
NW = NC * NS
BLK = 128
NB = N // BLK
TAIL = N - NB * BLK
SEG_PER_TILE = S // NS
BASE_BLOCKS = NB // NW
EXTRA_WORKERS = NB - BASE_BLOCKS * NW
MAXB = BASE_BLOCKS + 1


def _sc_partials(x, batch):
    mesh = plsc.VectorSubcoreMesh(core_axis_name="c", subcore_axis_name="s")

    @functools.partial(
        pl.kernel,
        out_type=[
            jax.ShapeDtypeStruct((NC, S, D), jnp.float32),
            jax.ShapeDtypeStruct((NW, S), jnp.float32),
        ],
        mesh=mesh,
        compiler_params=pltpu.CompilerParams(needs_layout_passes=False,
                                             use_tc_tiling_on_sc=False),
        scratch_types=[
            pltpu.VMEM((2, BLK, D), jnp.float32),
            pltpu.VMEM((MAXB, BLK), jnp.int32),
            pltpu.VMEM((TAIL, D), jnp.float32),
            pltpu.VMEM((TAIL,), jnp.int32),
            pltpu.VMEM((S,), jnp.float32),
            pltpu.VMEM((SEG_PER_TILE, D), jnp.float32),
            pltpu.VMEM_SHARED((S, D), jnp.float32),
            pltpu.SemaphoreType.DMA((2,)),
            pltpu.SemaphoreType.DMA,
            pltpu.SemaphoreType.DMA,
        ],
    )
    def sc_kernel(x_hbm, b_hbm, sum_out, cnt_out,
                  xbufs, ids_all, xt, ids_t, hist, zbuf, acc,
                  ld_sems, sc_sem, id_sem):
        c = lax.axis_index("c")
        s = lax.axis_index("s")
        wid = s * NC + c

        sb = BASE_BLOCKS * wid + jnp.minimum(wid, EXTRA_WORKERS)
        nblk = BASE_BLOCKS + jnp.where(wid < EXTRA_WORKERS, 1, 0)

        for k in range(MAXB):
            @pl.when(k < nblk)
            def _stage_ids():
                pltpu.async_copy(b_hbm.at[pl.ds((sb + k) * BLK, BLK)],
                                 ids_all.at[k], id_sem)

        for p in range(2):
            pltpu.async_copy(x_hbm.at[pl.ds((sb + p) * BLK, BLK)],
                             xbufs.at[p], ld_sems.at[p])

        zeros16 = jnp.zeros((16,), jnp.float32)

        def zrow(i, carry):
            def zcol(j, carry2):
                zbuf[i, pl.ds(j * 16, 16)] = zeros16
                return carry2
            return lax.fori_loop(0, D // 16, zcol, carry)
        lax.fori_loop(0, SEG_PER_TILE, zrow, 0)

        def zh(i, carry):
            hist[pl.ds(i * 16, 16)] = zeros16
            return carry
        lax.fori_loop(0, S // 16, zh, 0)

        pltpu.sync_copy(zbuf, acc.at[pl.ds(s * SEG_PER_TILE, SEG_PER_TILE)])
        plsc.subcore_barrier()

        for k in range(MAXB):
            @pl.when(k < nblk)
            def _drain_ids():
                pltpu.make_async_copy(b_hbm.at[pl.ds((sb + k) * BLK, BLK)],
                                      ids_all.at[k], id_sem).wait()

        lane = lax.iota(jnp.int32, 16)
        ones = jnp.full((16,), 1.0, jnp.float32)

        def pair(i, carry):
            for p in range(2):
                k = 2 * i + p

                @pl.when(k < nblk)
                def _block():
                    pltpu.make_async_copy(
                        x_hbm.at[pl.ds((sb + k) * BLK, BLK)],
                        xbufs.at[p], ld_sems.at[p]).wait()
                    h = pltpu.async_copy(xbufs.at[p], acc.at[ids_all.at[k]],
                                         sc_sem, add=True)

                    def grp(g, carry2):
                        idv = ids_all[k, pl.ds(g * 16, 16)]
                        plsc.addupdate_scatter(hist, [idv], ones)
                        return carry2
                    lax.fori_loop(0, BLK // 16, grp, 0)
                    h.wait()

                    @pl.when(k + 2 < nblk)
                    def _next_load():
                        pltpu.async_copy(
                            x_hbm.at[pl.ds((sb + k + 2) * BLK, BLK)],
                            xbufs.at[p], ld_sems.at[p])
            return carry
        lax.fori_loop(0, (MAXB + 1) // 2, pair, 0)

        @pl.when(wid == NW - 1)
        def _tail():
            base = NB * BLK
            pltpu.sync_copy(b_hbm.at[pl.ds(base, TAIL)], ids_t)
            pltpu.sync_copy(x_hbm.at[pl.ds(base, TAIL)], xt)
            pltpu.sync_copy(xt, acc.at[ids_t], add=True)

            def grp(g, carry):
                idv = ids_t[pl.ds(g * 16, 16)]
                plsc.addupdate_scatter(hist, [idv], ones)
                return carry
            lax.fori_loop(0, TAIL // 16, grp, 0)

        pltpu.sync_copy(hist, cnt_out.at[wid])
        plsc.subcore_barrier()
        pltpu.sync_copy(acc.at[pl.ds(s * SEG_PER_TILE, SEG_PER_TILE)],
                        sum_out.at[c, pl.ds(s * SEG_PER_TILE, SEG_PER_TILE)])

    return sc_kernel(x, batch)


def _combine(partial_sums, partial_counts):
    def body(sp_ref, cn_ref, o_ref):
        total = sp_ref[0] + sp_ref[1]
        cnt = jnp.maximum(jnp.sum(cn_ref[...], axis=0), 1.0)
        o_ref[...] = total / cnt[:, None]

    return pl.pallas_call(
        body,
        out_shape=jax.ShapeDtypeStruct((S, D), jnp.float32),
    )(partial_sums, partial_counts)


def kernel(x, batch):
    batch = batch.astype(jnp.int32)
    partial_sums, partial_counts = _sc_partials(x, batch)
    return _combine(partial_sums, partial_counts)
